# baseline (speedup 1.0000x reference)
import functools

import jax
import jax.numpy as jnp
from jax import lax
from jax.experimental import pallas as pl
from jax.experimental.pallas import tpu as pltpu
from jax.experimental.pallas import tpu_sc as plsc

_NC = 2
_NS = 16
_NW = _NC * _NS
_LANES = 16

_CHUNK = 64


def _fold_table_kernel(emb_ref, w_ref, b_ref, p_ref):
    p_ref[:, :] = (
        jnp.sum(emb_ref[:, :] * w_ref[:, :], axis=1, keepdims=True)
        + b_ref[:, :]
    )


def _fold_table(emb_table, W, b):
    V, D = emb_table.shape
    RB = 5000
    return pl.pallas_call(
        _fold_table_kernel,
        grid=(V // RB,),
        in_specs=[
            pl.BlockSpec((RB, D), lambda i: (i, 0)),
            pl.BlockSpec((1, D), lambda i: (0, 0)),
            pl.BlockSpec((1, 1), lambda i: (0, 0)),
        ],
        out_specs=pl.BlockSpec((RB, 1), lambda i: (i, 0)),
        out_shape=jax.ShapeDtypeStruct((V, 1), jnp.float32),
    )(emb_table, W, b.reshape(1, 1))


def _make_sc_pool(V, B, S):
    nblk = B // _CHUNK
    blk_per_w = nblk // _NW
    groups = _CHUNK // _LANES
    inv_s = 1.0 / S
    mesh = plsc.VectorSubcoreMesh(
        core_axis_name="c", subcore_axis_name="s",
        num_cores=_NC, num_subcores=_NS,
    )

    unroll = 4
    assert S % unroll == 0

    @functools.partial(
        pl.kernel,
        out_type=jax.ShapeDtypeStruct((B,), jnp.float32),
        mesh=mesh,
        scratch_types=[
            pltpu.VMEM((V,), jnp.float32),
            pltpu.VMEM((_CHUNK, S), jnp.int32),
            pltpu.VMEM((_CHUNK,), jnp.float32),
        ],
        compiler_params=pltpu.CompilerParams(needs_layout_passes=False),
    )
    def sc_pool(p_hbm, idx_hbm, out_hbm, p_v, chunk_v, out_v):
        wid = lax.axis_index("s") * _NC + lax.axis_index("c")
        pltpu.sync_copy(p_hbm, p_v)
        lanes = lax.iota(jnp.int32, _LANES)
        rows = [lanes + g * _LANES for g in range(groups)]

        def blk_body(i, carry):
            blk = wid * blk_per_w + i
            pltpu.sync_copy(idx_hbm.at[blk], chunk_v)

            def s_body(s, accs):
                accs = list(accs)
                for j in range(unroll):
                    col = jnp.full((_LANES,), s * unroll + j, jnp.int32)
                    for g in range(groups):
                        ids = plsc.load_gather(chunk_v, [rows[g], col])
                        accs[g] = accs[g] + plsc.load_gather(p_v, [ids])
                return tuple(accs)

            zero = jnp.zeros((_LANES,), jnp.float32)
            accs = lax.fori_loop(0, S // unroll, s_body, (zero,) * groups)
            for g in range(groups):
                z = accs[g] * inv_s
                out_v[pl.ds(g * _LANES, _LANES)] = 1.0 / (1.0 + jnp.exp(-z))
            pltpu.sync_copy(out_v, out_hbm.at[pl.ds(blk * _CHUNK, _CHUNK)])
            return carry

        lax.fori_loop(0, blk_per_w, blk_body, 0)

    return sc_pool


def kernel(text, emb_table, W, b):
    B, S = text.shape
    V, _D = emb_table.shape
    p = _fold_table(emb_table, W, b).reshape(V)
    v_pad = -(-V // 128) * 128
    p = jnp.pad(p, (0, v_pad - V))
    idx_blocks = text.reshape(B // _CHUNK, _CHUNK, S)
    out = _make_sc_pool(v_pad, B, S)(p, idx_blocks)
    return out.reshape(B, 1)


# speedup vs baseline: 1.2835x; 1.2835x over previous
import functools

import jax
import jax.numpy as jnp
from jax import lax
from jax.experimental import pallas as pl
from jax.experimental.pallas import tpu as pltpu
from jax.experimental.pallas import tpu_sc as plsc

_NC = 2
_NS = 16
_NW = _NC * _NS
_LANES = 16

_CHUNK = 64


def _fold_table_kernel(emb_ref, w_ref, b_ref, p_ref):
    p_ref[...] = (
        jnp.sum(emb_ref[...] * w_ref[...], axis=1, keepdims=True)
        + b_ref[...]
    )


def _fold_table(emb_table, W, b):
    V, D = emb_table.shape
    RB = 5000
    return pl.pallas_call(
        _fold_table_kernel,
        grid=(V // RB,),
        in_specs=[
            pl.BlockSpec((RB, D), lambda i: (i, 0)),
            pl.BlockSpec((1, D), lambda i: (0, 0)),
            pl.BlockSpec((1, 1), lambda i: (0, 0)),
        ],
        out_specs=pl.BlockSpec((RB, 1), lambda i: (i, 0)),
        out_shape=jax.ShapeDtypeStruct((V, 1), jnp.float32),
    )(emb_table, W, b.reshape(1, 1))


def _make_sc_pool(V, B, S):
    rows_per_w = B // _NW
    n_chunks = rows_per_w // _CHUNK
    n_full = S // _LANES
    tail = S - n_full * _LANES
    groups = _CHUNK // _LANES
    inv_s = 1.0 / S
    mesh = plsc.VectorSubcoreMesh(
        core_axis_name="c", subcore_axis_name="s",
        num_cores=_NC, num_subcores=_NS,
    )

    @functools.partial(
        pl.kernel,
        out_type=jax.ShapeDtypeStruct((B,), jnp.float32),
        mesh=mesh,
        scratch_types=[
            pltpu.VMEM((V,), jnp.float32),
            pltpu.VMEM((_CHUNK, S), jnp.int32),
            pltpu.VMEM((_CHUNK,), jnp.float32),
        ],
        compiler_params=pltpu.CompilerParams(needs_layout_passes=False),
    )
    def sc_pool(p_hbm, idx_hbm, out_hbm, p_v, chunk_v, out_v):
        wid = lax.axis_index("s") * _NC + lax.axis_index("c")
        pltpu.sync_copy(p_hbm, p_v)
        zero = jnp.zeros((_LANES,), jnp.float32)
        lanes = lax.iota(jnp.int32, _LANES)
        last_mask = lanes == (_LANES - 1)
        if tail:
            tmask = lanes >= (_LANES - tail)

        def chunk_body(c, carry):
            base = (wid * n_chunks + c) * _CHUNK
            pltpu.sync_copy(idx_hbm.at[pl.ds(base, _CHUNK)], chunk_v)

            def row_body(r, carry2):
                accs = [zero] * 4
                for j in range(n_full):
                    ids = chunk_v[r, pl.ds(j * _LANES, _LANES)]
                    accs[j % 4] = accs[j % 4] + plsc.load_gather(p_v, [ids])
                if tail:
                    ids = chunk_v[r, pl.ds(S - _LANES, _LANES)]
                    vals = plsc.load_gather(p_v, [ids])
                    accs[n_full % 4] = accs[n_full % 4] + jnp.where(
                        tmask, vals, 0.0
                    )
                acc = (accs[0] + accs[1]) + (accs[2] + accs[3])
                cs = plsc.cumsum(acc)
                plsc.store_scatter(
                    out_v, [jnp.full((_LANES,), r, jnp.int32)], cs,
                    mask=last_mask,
                )
                return carry2

            lax.fori_loop(0, _CHUNK, row_body, 0)
            for g in range(groups):
                z = out_v[pl.ds(g * _LANES, _LANES)] * inv_s
                out_v[pl.ds(g * _LANES, _LANES)] = 1.0 / (1.0 + jnp.exp(-z))
            pltpu.sync_copy(out_v, out_hbm.at[pl.ds(base, _CHUNK)])
            return carry

        lax.fori_loop(0, n_chunks, chunk_body, 0)

    return sc_pool


def kernel(text, emb_table, W, b):
    B, S = text.shape
    V, _D = emb_table.shape
    p = _fold_table(emb_table, W, b).reshape(V)
    out = _make_sc_pool(V, B, S)(p, text)
    return out.reshape(B, 1)


# speedup vs baseline: 1.3907x; 1.0835x over previous
import functools

import jax
import jax.numpy as jnp
from jax import lax
from jax.experimental import pallas as pl
from jax.experimental.pallas import tpu as pltpu
from jax.experimental.pallas import tpu_sc as plsc

_NC = 2
_NS = 16
_NW = _NC * _NS
_LANES = 16

_CHUNK = 64


def _fold_table_kernel(emb_ref, w_ref, b_ref, p_ref):
    p_ref[...] = (
        jnp.sum(emb_ref[...] * w_ref[...], axis=1, keepdims=True)
        + b_ref[...]
    )


def _fold_table(emb_table, W, b):
    V, D = emb_table.shape
    RB = 10000
    return pl.pallas_call(
        _fold_table_kernel,
        grid=(V // RB,),
        in_specs=[
            pl.BlockSpec((RB, D), lambda i: (i, 0)),
            pl.BlockSpec((1, D), lambda i: (0, 0)),
            pl.BlockSpec((1, 1), lambda i: (0, 0)),
        ],
        out_specs=pl.BlockSpec((RB, 1), lambda i: (i, 0)),
        out_shape=jax.ShapeDtypeStruct((V, 1), jnp.float32),
    )(emb_table, W, b.reshape(1, 1))


def _make_sc_pool(V, B, S):
    rows_per_w = B // _NW
    n_chunks = rows_per_w // _CHUNK
    n_full = S // _LANES
    tail = S - n_full * _LANES
    groups = _CHUNK // _LANES
    inv_s = 1.0 / S
    mesh = plsc.VectorSubcoreMesh(
        core_axis_name="c", subcore_axis_name="s",
        num_cores=_NC, num_subcores=_NS,
    )

    @functools.partial(
        pl.kernel,
        out_type=jax.ShapeDtypeStruct((B,), jnp.float32),
        mesh=mesh,
        scratch_types=[
            pltpu.VMEM((V,), jnp.float32),
            pltpu.VMEM((_CHUNK, S), jnp.int32),
            pltpu.VMEM((_CHUNK,), jnp.float32),
        ],
        compiler_params=pltpu.CompilerParams(needs_layout_passes=False),
    )
    def sc_pool(p_hbm, idx_hbm, out_hbm, p_v, chunk_v, out_v):
        wid = lax.axis_index("s") * _NC + lax.axis_index("c")
        pltpu.sync_copy(p_hbm, p_v)
        zero = jnp.zeros((_LANES,), jnp.float32)
        lanes = lax.iota(jnp.int32, _LANES)
        last_mask = lanes == (_LANES - 1)
        if tail:
            tmask = lanes >= (_LANES - tail)

        def chunk_body(c, carry):
            base = (wid * n_chunks + c) * _CHUNK
            pltpu.sync_copy(idx_hbm.at[pl.ds(base, _CHUNK)], chunk_v)

            @plsc.parallel_loop(0, _CHUNK, unroll=2)
            def row_body(r):
                accs = [zero] * 4
                for j in range(n_full):
                    ids = chunk_v[r, pl.ds(j * _LANES, _LANES)]
                    accs[j % 4] = accs[j % 4] + plsc.load_gather(p_v, [ids])
                if tail:
                    ids = chunk_v[r, pl.ds(S - _LANES, _LANES)]
                    vals = plsc.load_gather(p_v, [ids])
                    accs[n_full % 4] = accs[n_full % 4] + jnp.where(
                        tmask, vals, 0.0
                    )
                acc = (accs[0] + accs[1]) + (accs[2] + accs[3])
                cs = plsc.cumsum(acc)
                plsc.store_scatter(
                    out_v, [jnp.full((_LANES,), r, jnp.int32)], cs,
                    mask=last_mask,
                )
            for g in range(groups):
                z = out_v[pl.ds(g * _LANES, _LANES)] * inv_s
                out_v[pl.ds(g * _LANES, _LANES)] = 1.0 / (1.0 + jnp.exp(-z))
            pltpu.sync_copy(out_v, out_hbm.at[pl.ds(base, _CHUNK)])
            return carry

        lax.fori_loop(0, n_chunks, chunk_body, 0)

    return sc_pool


def kernel(text, emb_table, W, b):
    B, S = text.shape
    V, _D = emb_table.shape
    p = _fold_table(emb_table, W, b).reshape(V)
    out = _make_sc_pool(V, B, S)(p, text)
    return out.reshape(B, 1)


# speedup vs baseline: 2.5747x; 1.8513x over previous
import functools

import jax
import jax.numpy as jnp
from jax import lax
from jax.experimental import pallas as pl
from jax.experimental.pallas import tpu as pltpu
from jax.experimental.pallas import tpu_sc as plsc

_NC = 2
_NS = 16
_NW = _NC * _NS
_LANES = 16

_CHUNK = 128
_RB = 20480


def _fold_table_kernel(emb_ref, w_ref, b_ref, p_ref):
    p_ref[...] = lax.dot_general(
        w_ref[...], emb_ref[...],
        dimension_numbers=(((1,), (1,)), ((), ())),
        preferred_element_type=jnp.float32,
    ) + b_ref[...]


def _fold_table(emb_table, W, b, v_pad):
    V, D = emb_table.shape
    return pl.pallas_call(
        _fold_table_kernel,
        grid=(v_pad // _RB,),
        in_specs=[
            pl.BlockSpec((_RB, D), lambda i: (i, 0)),
            pl.BlockSpec((1, D), lambda i: (0, 0)),
            pl.BlockSpec((1, 1), lambda i: (0, 0)),
        ],
        out_specs=pl.BlockSpec((1, _RB), lambda i: (0, i)),
        out_shape=jax.ShapeDtypeStruct((1, v_pad), jnp.float32),
    )(emb_table, W, b.reshape(1, 1))


def _make_sc_pool(V, B, S):
    cols_per_w = B // _NW
    n_chunks = cols_per_w // _CHUNK
    groups = _CHUNK // _LANES
    inv_s = 1.0 / S
    mesh = plsc.VectorSubcoreMesh(
        core_axis_name="c", subcore_axis_name="s",
        num_cores=_NC, num_subcores=_NS,
    )

    @functools.partial(
        pl.kernel,
        out_type=jax.ShapeDtypeStruct((B,), jnp.float32),
        mesh=mesh,
        scratch_types=[
            pltpu.VMEM((V,), jnp.float32),
            pltpu.VMEM((S, _CHUNK), jnp.int32),
            pltpu.VMEM((_CHUNK,), jnp.float32),
        ],
        compiler_params=pltpu.CompilerParams(needs_layout_passes=False),
    )
    def sc_pool(p_hbm, idx_hbm, out_hbm, p_v, chunk_v, out_v):
        wid = lax.axis_index("s") * _NC + lax.axis_index("c")
        pltpu.sync_copy(p_hbm, p_v)
        zero = jnp.zeros((_LANES,), jnp.float32)

        def chunk_body(c, carry):
            base = (wid * n_chunks + c) * _CHUNK
            pltpu.sync_copy(idx_hbm.at[:, pl.ds(base, _CHUNK)], chunk_v)

            def s_body(s2, accs):
                out = list(accs)
                for j in range(2):
                    s = s2 * 2 + j
                    for g in range(groups):
                        ids = chunk_v[s, pl.ds(g * _LANES, _LANES)]
                        out[g] = out[g] + plsc.load_gather(p_v, [ids])
                return tuple(out)

            accs = lax.fori_loop(0, S // 2, s_body, (zero,) * groups)
            for g in range(groups):
                z = accs[g] * inv_s
                out_v[pl.ds(g * _LANES, _LANES)] = 1.0 / (1.0 + jnp.exp(-z))
            pltpu.sync_copy(out_v, out_hbm.at[pl.ds(base, _CHUNK)])
            return carry

        lax.fori_loop(0, n_chunks, chunk_body, 0)

    return sc_pool


def kernel(text, emb_table, W, b):
    B, S = text.shape
    V, _D = emb_table.shape
    v_pad = -(-V // _RB) * _RB
    p = _fold_table(emb_table, W, b, v_pad).reshape(v_pad)
    out = _make_sc_pool(v_pad, B, S)(p, text.T)
    return out.reshape(B, 1)
